# Initial kernel scaffold; baseline (speedup 1.0000x reference)
import jax
import jax.numpy as jnp
from jax.experimental import pallas as pl


def kernel(theta):
    raise NotImplementedError("write your pallas kernel here")


# speedup vs baseline: 4.0413x; 4.0413x over previous
import jax
import jax.numpy as jnp
from jax.experimental import pallas as pl
from jax.experimental.pallas import tpu as pltpu

N = 16
D = 1024


def _fwd_body(theta_ref, mf_ref, carry):
    s = pl.program_id(0)

    @pl.when(s == 0)
    def _():
        carry[...] = jnp.zeros((D,), jnp.float32)

    t = theta_ref[0] + carry[...][:, None]
    c = jnp.max(t, axis=0)
    sm = jnp.sum(jnp.exp(t - c[None, :]), axis=0)
    new = c + jnp.log(sm)
    mf_ref[0, 0] = new
    carry[...] = new


_fwd = pl.pallas_call(
    _fwd_body,
    grid=(N - 1,),
    in_specs=[pl.BlockSpec((1, D, D), lambda s: (s, 0, 0))],
    out_specs=pl.BlockSpec((1, 1, D), lambda s: (s, 0, 0)),
    out_shape=jax.ShapeDtypeStruct((N - 1, 1, D), jnp.float32),
    scratch_shapes=[pltpu.VMEM((D,), jnp.float32)],
)


def _bwd_body(theta_ref, mf_ref, out_ref, carry):
    s = pl.program_id(0)

    @pl.when(s == 0)
    def _():
        carry[...] = jnp.zeros((D,), jnp.float32)

    t1 = theta_ref[0] + carry[...][None, :]
    r = jnp.max(t1, axis=1)
    sm = jnp.sum(jnp.exp(t1 - r[:, None]), axis=1)
    lse1 = r + jnp.log(sm)
    mf = mf_ref[0, 0]
    q = mf + lse1
    qm = jnp.max(q)
    z = qm + jnp.log(jnp.sum(jnp.exp(q - qm)))
    out_ref[0] = t1 + (mf - z)[:, None]
    carry[...] = lse1


_bwd = pl.pallas_call(
    _bwd_body,
    grid=(N,),
    in_specs=[
        pl.BlockSpec((1, D, D), lambda s: (N - 1 - s, 0, 0)),
        pl.BlockSpec((1, 1, D), lambda s: (N - 1 - s, 0, 0)),
    ],
    out_specs=pl.BlockSpec((1, D, D), lambda s: (N - 1 - s, 0, 0)),
    out_shape=jax.ShapeDtypeStruct((N, D, D), jnp.float32),
    scratch_shapes=[pltpu.VMEM((D,), jnp.float32)],
)


def kernel(theta):
    mf_tail = _fwd(theta)
    mf = jnp.concatenate([jnp.zeros((1, 1, D), jnp.float32), mf_tail], axis=0)
    return _bwd(theta, mf)
